# baseline (speedup 1.0000x reference)
import functools

import jax
import jax.numpy as jnp
from jax import lax
from jax.experimental import pallas as pl
from jax.experimental.pallas import tpu as pltpu
from jax.experimental.pallas import tpu_sc as plsc

VOCAB = 1000000
EMBED = 64
CTX = 200
BATCH = 4096

NC, NS = 2, 16
NW = NC * NS
ITEMS_PER_W = BATCH // NW
L = 16

NB = 4
D = 2
GROUPS = ITEMS_PER_W // NB


def _pe_table():
    positions = jnp.arange(CTX, dtype=jnp.float32)
    dimensions = jnp.arange(EMBED, dtype=jnp.float32)
    exponent = (dimensions // 2) * 2.0 / EMBED
    divisor = jnp.power(10000.0, exponent)
    angle_rates = positions[:, None] / divisor
    pe = jnp.zeros_like(angle_rates)
    pe = pe.at[:, 0::2].set(jnp.sin(angle_rates[:, 0::2]))
    pe = pe.at[:, 1::2].set(jnp.cos(angle_rates[:, 1::2]))
    return pe


@functools.partial(
    pl.kernel,
    out_type=jax.ShapeDtypeStruct((BATCH * CTX, 2 * EMBED), jnp.float32),
    mesh=plsc.VectorSubcoreMesh(
        core_axis_name="c", subcore_axis_name="s", num_cores=NC, num_subcores=NS
    ),
    scratch_types=[
        pltpu.VMEM((NB, CTX), jnp.int32),
        pltpu.VMEM((NB, CTX, 2 * EMBED), jnp.float32),
        pltpu.VMEM((CTX, EMBED), jnp.float32),
        pltpu.SemaphoreType.DMA((NB,)),
        pltpu.SemaphoreType.DMA((NB,)),
    ],
    compiler_params=pltpu.CompilerParams(use_tc_tiling_on_sc=False),
)
def _gather_add_pe(x_hbm, pe_hbm, table_hbm, out_hbm, idx_v, rows_v, pe_v, gsem, ssem):
    wid = lax.axis_index("s") * NC + lax.axis_index("c")
    pltpu.sync_copy(pe_hbm, pe_v)
    base = wid * ITEMS_PER_W

    def issue_gather(item, b):
        row0 = (base + item) * CTX
        pltpu.sync_copy(x_hbm.at[pl.ds(row0, CTX)], idx_v.at[b])
        pltpu.async_copy(
            table_hbm.at[idx_v.at[b, pl.ds(0, 128)]],
            rows_v.at[b, pl.ds(0, 128)],
            gsem.at[b],
        )
        pltpu.async_copy(
            table_hbm.at[idx_v.at[b, pl.ds(128, CTX - 128)]],
            rows_v.at[b, pl.ds(128, CTX - 128)],
            gsem.at[b],
        )

    def wait_gather(b):
        pltpu.make_async_copy(
            table_hbm.at[idx_v.at[b]], rows_v.at[b], gsem.at[b]
        ).wait()

    def wait_store(b):
        pltpu.make_async_copy(
            rows_v.at[b, pl.ds(0, CTX), pl.ds(0, EMBED)],
            out_hbm.at[pl.ds(0, CTX), pl.ds(0, EMBED)],
            ssem.at[b],
        ).wait()

    def add_pe(b):
        def body(r, carry):
            for rr in range(2):
                for c in range(EMBED // L):
                    sl = pl.ds(c * L, L)
                    plsc.addupdate(rows_v.at[b, 2 * r + rr, sl], pe_v[2 * r + rr, sl])
            return carry

        lax.fori_loop(0, CTX // 2, body, 0)

    def issue_store(item, b):
        row0 = (base + item) * CTX
        pltpu.async_copy(
            rows_v.at[b, pl.ds(0, CTX), pl.ds(0, EMBED)],
            out_hbm.at[pl.ds(row0, CTX), pl.ds(0, EMBED)],
            ssem.at[b],
        )

    def process(item, b, do_wait_store, do_issue_gather):
        wait_gather(b)
        add_pe(b)
        issue_store(item, b)
        bn = (b + D) % NB
        if do_issue_gather:
            if do_wait_store:
                wait_store(bn)
            issue_gather(item + D, bn)

    for b in range(D):
        issue_gather(b, b)

    for b in range(NB):
        process(b, b, do_wait_store=(b >= NB - D), do_issue_gather=True)

    def group(j, carry):
        i0 = j * NB
        for b in range(NB):
            process(i0 + b, b, do_wait_store=True, do_issue_gather=True)
        return carry

    lax.fori_loop(1, GROUPS - 1, group, 0)

    i0 = (GROUPS - 1) * NB
    for b in range(NB):
        process(
            i0 + b, b, do_wait_store=True, do_issue_gather=(i0 + b + D < ITEMS_PER_W)
        )

    for b in range(NB):
        wait_store(b)


def kernel(x, table):
    pe = _pe_table()
    xf = x.reshape(-1).astype(jnp.int32)
    table_p = jnp.pad(table, ((0, 0), (0, 2 * EMBED - EMBED)))
    out = _gather_add_pe(xf, pe, table_p)
    return out[:, :EMBED].reshape(BATCH, CTX, EMBED)


# speedup vs baseline: 1.5034x; 1.5034x over previous
import functools

import jax
import jax.numpy as jnp
from jax import lax
from jax.experimental import pallas as pl
from jax.experimental.pallas import tpu as pltpu
from jax.experimental.pallas import tpu_sc as plsc

VOCAB = 1000000
EMBED = 64
CTX = 200
BATCH = 4096

NC, NS = 2, 16
NW = NC * NS
ITEMS_PER_W = BATCH // NW
L = 16

NB = 8
D = 4
GROUPS = ITEMS_PER_W // NB
IDXPAD = 208

DW = 1024
DGRID = (VOCAB + 2 * DW - 1) // (2 * DW)
VOCAB_PAD = DGRID * 2 * DW


def _pe_table():
    positions = jnp.arange(CTX, dtype=jnp.float32)
    dimensions = jnp.arange(EMBED, dtype=jnp.float32)
    exponent = (dimensions // 2) * 2.0 / EMBED
    divisor = jnp.power(10000.0, exponent)
    angle_rates = positions[:, None] / divisor
    pe = jnp.zeros_like(angle_rates)
    pe = pe.at[:, 0::2].set(jnp.sin(angle_rates[:, 0::2]))
    pe = pe.at[:, 1::2].set(jnp.cos(angle_rates[:, 1::2]))
    return pe


def _detile_body(a_ref, b_ref, o_ref):
    o_ref[:, 0:EMBED] = a_ref[...].T
    o_ref[:, EMBED : 2 * EMBED] = b_ref[...].T


_detile = pl.pallas_call(
    _detile_body,
    out_shape=jax.ShapeDtypeStruct((VOCAB_PAD // 2, 2 * EMBED), jnp.float32),
    grid=(DGRID,),
    in_specs=[
        pl.BlockSpec((EMBED, DW), lambda g: (0, 2 * g)),
        pl.BlockSpec(
            (EMBED, DW), lambda g: (0, jnp.minimum(2 * g + 1, VOCAB // DW - 1))
        ),
    ],
    out_specs=pl.BlockSpec((DW, 2 * EMBED), lambda g: (g, 0)),
)


@functools.partial(
    pl.kernel,
    out_type=jax.ShapeDtypeStruct((BATCH * CTX, 2 * EMBED), jnp.float32),
    mesh=plsc.VectorSubcoreMesh(
        core_axis_name="c", subcore_axis_name="s", num_cores=NC, num_subcores=NS
    ),
    scratch_types=[
        pltpu.VMEM((NB, IDXPAD), jnp.int32),
        pltpu.VMEM((NB, CTX, EMBED), jnp.float32),
        pltpu.VMEM((CTX, EMBED), jnp.float32),
        pltpu.SemaphoreType.DMA((NB,)),
        pltpu.SemaphoreType.DMA((NB,)),
    ],
    compiler_params=pltpu.CompilerParams(use_tc_tiling_on_sc=False),
)
def _gather_add_pe(x_hbm, pe_hbm, table_hbm, out_hbm, idx_v, rows_v, pe_v, gsem, ssem):
    wid = lax.axis_index("s") * NC + lax.axis_index("c")
    pltpu.sync_copy(pe_hbm, pe_v)
    base = wid * ITEMS_PER_W

    def issue_gather(item, b):
        row0 = (base + item) * CTX
        pltpu.sync_copy(x_hbm.at[pl.ds(row0, CTX)], idx_v.at[b, pl.ds(0, CTX)])
        for c in range(IDXPAD // L):
            sl = pl.ds(c * L, L)
            t = idx_v[b, sl]
            j = (
                (t & jnp.int32(~2047))
                + ((t & jnp.int32(1023)) << 1)
                + ((t >> 10) & jnp.int32(1))
            )
            idx_v[b, sl] = j
        pltpu.async_copy(
            table_hbm.at[idx_v.at[b, pl.ds(0, 128)]],
            rows_v.at[b, pl.ds(0, 128)],
            gsem.at[b],
        )
        pltpu.async_copy(
            table_hbm.at[idx_v.at[b, pl.ds(128, CTX - 128)]],
            rows_v.at[b, pl.ds(128, CTX - 128)],
            gsem.at[b],
        )

    def wait_gather(b):
        pltpu.make_async_copy(
            table_hbm.at[idx_v.at[b, pl.ds(0, CTX)]], rows_v.at[b], gsem.at[b]
        ).wait()

    def wait_store(b):
        pltpu.make_async_copy(
            rows_v.at[b], out_hbm.at[pl.ds(0, CTX), pl.ds(0, EMBED)], ssem.at[b]
        ).wait()

    def add_pe(b):
        def body(r, carry):
            for rr in range(2):
                for c in range(EMBED // L):
                    sl = pl.ds(c * L, L)
                    plsc.addupdate(rows_v.at[b, 2 * r + rr, sl], pe_v[2 * r + rr, sl])
            return carry

        lax.fori_loop(0, CTX // 2, body, 0)

    def issue_store(item, b):
        row0 = (base + item) * CTX
        pltpu.async_copy(
            rows_v.at[b], out_hbm.at[pl.ds(row0, CTX), pl.ds(0, EMBED)], ssem.at[b]
        )

    def process(item, b, do_wait_store, do_issue_gather):
        wait_gather(b)
        add_pe(b)
        issue_store(item, b)
        bn = (b + D) % NB
        if do_issue_gather:
            if do_wait_store:
                wait_store(bn)
            issue_gather(item + D, bn)

    for b in range(D):
        issue_gather(b, b)

    for b in range(NB):
        process(b, b, do_wait_store=(b >= NB - D), do_issue_gather=True)

    def group(j, carry):
        i0 = j * NB
        for b in range(NB):
            process(i0 + b, b, do_wait_store=True, do_issue_gather=True)
        return carry

    lax.fori_loop(1, GROUPS - 1, group, 0)

    i0 = (GROUPS - 1) * NB
    for b in range(NB):
        process(
            i0 + b, b, do_wait_store=True, do_issue_gather=(i0 + b + D < ITEMS_PER_W)
        )

    for b in range(NB):
        wait_store(b)


def kernel(x, table):
    pe = _pe_table()
    xf = x.reshape(-1).astype(jnp.int32)
    tt = table.T
    table_lin = _detile(tt, tt).reshape(VOCAB_PAD, EMBED)
    out = _gather_add_pe(xf, pe, table_lin)
    return out[:, :EMBED].reshape(BATCH, CTX, EMBED)


# speedup vs baseline: 1.6193x; 1.0771x over previous
import functools

import jax
import jax.numpy as jnp
from jax import lax
from jax.experimental import pallas as pl
from jax.experimental.pallas import tpu as pltpu
from jax.experimental.pallas import tpu_sc as plsc

VOCAB = 1000000
EMBED = 64
CTX = 200
BATCH = 4096

NC, NS = 2, 16
NW = NC * NS
ITEMS_PER_W = BATCH // NW
L = 16

NB = 8
D = 4
GROUPS = ITEMS_PER_W // NB
IDXPAD = 208

DW = 1024
DGRID = (VOCAB + 2 * DW - 1) // (2 * DW)
VOCAB_PAD = DGRID * 2 * DW


def _pe_table():
    positions = jnp.arange(CTX, dtype=jnp.float32)
    dimensions = jnp.arange(EMBED, dtype=jnp.float32)
    exponent = (dimensions // 2) * 2.0 / EMBED
    divisor = jnp.power(10000.0, exponent)
    angle_rates = positions[:, None] / divisor
    pe = jnp.zeros_like(angle_rates)
    pe = pe.at[:, 0::2].set(jnp.sin(angle_rates[:, 0::2]))
    pe = pe.at[:, 1::2].set(jnp.cos(angle_rates[:, 1::2]))
    return pe


def _detile_body(a_ref, b_ref, o_ref):
    o_ref[...] = jnp.concatenate([a_ref[...], b_ref[...]], axis=0).T


_detile = pl.pallas_call(
    _detile_body,
    out_shape=jax.ShapeDtypeStruct((VOCAB_PAD // 2, 2 * EMBED), jnp.float32),
    grid=(DGRID,),
    in_specs=[
        pl.BlockSpec((EMBED, DW), lambda g: (0, 2 * g)),
        pl.BlockSpec(
            (EMBED, DW), lambda g: (0, jnp.minimum(2 * g + 1, VOCAB // DW - 1))
        ),
    ],
    out_specs=pl.BlockSpec((DW, 2 * EMBED), lambda g: (g, 0)),
)


@functools.partial(
    pl.kernel,
    out_type=jax.ShapeDtypeStruct((BATCH * CTX, 2 * EMBED), jnp.float32),
    mesh=plsc.VectorSubcoreMesh(
        core_axis_name="c", subcore_axis_name="s", num_cores=NC, num_subcores=NS
    ),
    scratch_types=[
        pltpu.VMEM((NB, IDXPAD), jnp.int32),
        pltpu.VMEM((NB, CTX, EMBED), jnp.float32),
        pltpu.VMEM((CTX, EMBED), jnp.float32),
        pltpu.SemaphoreType.DMA((NB,)),
        pltpu.SemaphoreType.DMA((NB,)),
    ],
    compiler_params=pltpu.CompilerParams(use_tc_tiling_on_sc=False),
)
def _gather_add_pe(x_hbm, pe_hbm, table_hbm, out_hbm, idx_v, rows_v, pe_v, gsem, ssem):
    wid = lax.axis_index("s") * NC + lax.axis_index("c")
    pltpu.sync_copy(pe_hbm, pe_v)
    base = wid * ITEMS_PER_W

    def issue_gather(item, b):
        row0 = (base + item) * CTX
        pltpu.sync_copy(x_hbm.at[pl.ds(row0, CTX)], idx_v.at[b, pl.ds(0, CTX)])
        for c in range(IDXPAD // L):
            sl = pl.ds(c * L, L)
            t = idx_v[b, sl]
            j = (
                (t & jnp.int32(~2047))
                + ((t & jnp.int32(1023)) << 1)
                + ((t >> 10) & jnp.int32(1))
            )
            idx_v[b, sl] = j
        pltpu.async_copy(
            table_hbm.at[idx_v.at[b, pl.ds(0, 128)]],
            rows_v.at[b, pl.ds(0, 128)],
            gsem.at[b],
        )
        pltpu.async_copy(
            table_hbm.at[idx_v.at[b, pl.ds(128, CTX - 128)]],
            rows_v.at[b, pl.ds(128, CTX - 128)],
            gsem.at[b],
        )

    def wait_gather(b):
        pltpu.make_async_copy(
            table_hbm.at[idx_v.at[b, pl.ds(0, CTX)]], rows_v.at[b], gsem.at[b]
        ).wait()

    def wait_store(b):
        pltpu.make_async_copy(
            rows_v.at[b], out_hbm.at[pl.ds(0, CTX), pl.ds(0, EMBED)], ssem.at[b]
        ).wait()

    def add_pe(b):
        def body(r, carry):
            for rr in range(2):
                for c in range(EMBED // L):
                    sl = pl.ds(c * L, L)
                    plsc.addupdate(rows_v.at[b, 2 * r + rr, sl], pe_v[2 * r + rr, sl])
            return carry

        lax.fori_loop(0, CTX // 2, body, 0)

    def issue_store(item, b):
        row0 = (base + item) * CTX
        pltpu.async_copy(
            rows_v.at[b], out_hbm.at[pl.ds(row0, CTX), pl.ds(0, EMBED)], ssem.at[b]
        )

    def process(item, b, do_wait_store, do_issue_gather):
        wait_gather(b)
        add_pe(b)
        issue_store(item, b)
        bn = (b + D) % NB
        if do_issue_gather:
            if do_wait_store:
                wait_store(bn)
            issue_gather(item + D, bn)

    for b in range(D):
        issue_gather(b, b)

    for b in range(NB):
        process(b, b, do_wait_store=(b >= NB - D), do_issue_gather=True)

    def group(j, carry):
        i0 = j * NB
        for b in range(NB):
            process(i0 + b, b, do_wait_store=True, do_issue_gather=True)
        return carry

    lax.fori_loop(1, GROUPS - 1, group, 0)

    i0 = (GROUPS - 1) * NB
    for b in range(NB):
        process(
            i0 + b, b, do_wait_store=True, do_issue_gather=(i0 + b + D < ITEMS_PER_W)
        )

    for b in range(NB):
        wait_store(b)


def kernel(x, table):
    pe = _pe_table()
    xf = x.reshape(-1).astype(jnp.int32)
    tt = table.T
    table_lin = _detile(tt, tt).reshape(VOCAB_PAD, EMBED)
    out = _gather_add_pe(xf, pe, table_lin)
    return out[:, :EMBED].reshape(BATCH, CTX, EMBED)


# speedup vs baseline: 1.8903x; 1.1674x over previous
import functools

import jax
import jax.numpy as jnp
from jax import lax
from jax.experimental import pallas as pl
from jax.experimental.pallas import tpu as pltpu
from jax.experimental.pallas import tpu_sc as plsc

VOCAB = 1000000
EMBED = 64
CTX = 200
BATCH = 4096

NC, NS = 2, 16
NW = NC * NS
ITEMS_PER_W = BATCH // NW
L = 16

NB = 8
D = 4
GROUPS = ITEMS_PER_W // NB
IDXPAD = 208

DW = 2048
LOG2DW = DW.bit_length() - 1
DGRID = (VOCAB + 2 * DW - 1) // (2 * DW)
VOCAB_PAD = DGRID * 2 * DW


def _pe_table():
    positions = jnp.arange(CTX, dtype=jnp.float32)
    dimensions = jnp.arange(EMBED, dtype=jnp.float32)
    exponent = (dimensions // 2) * 2.0 / EMBED
    divisor = jnp.power(10000.0, exponent)
    angle_rates = positions[:, None] / divisor
    pe = jnp.zeros_like(angle_rates)
    pe = pe.at[:, 0::2].set(jnp.sin(angle_rates[:, 0::2]))
    pe = pe.at[:, 1::2].set(jnp.cos(angle_rates[:, 1::2]))
    return pe


def _detile_body(a_ref, b_ref, o_ref):
    o_ref[...] = jnp.concatenate([a_ref[...], b_ref[...]], axis=0).T


_detile = pl.pallas_call(
    _detile_body,
    out_shape=jax.ShapeDtypeStruct((VOCAB_PAD // 2, 2 * EMBED), jnp.float32),
    grid=(DGRID,),
    in_specs=[
        pl.BlockSpec((EMBED, DW), lambda g: (0, 2 * g)),
        pl.BlockSpec(
            (EMBED, DW), lambda g: (0, jnp.minimum(2 * g + 1, VOCAB // DW - 1))
        ),
    ],
    out_specs=pl.BlockSpec((DW, 2 * EMBED), lambda g: (g, 0)),
)


@functools.partial(
    pl.kernel,
    out_type=jax.ShapeDtypeStruct((BATCH * CTX, 2 * EMBED), jnp.float32),
    mesh=plsc.VectorSubcoreMesh(
        core_axis_name="c", subcore_axis_name="s", num_cores=NC, num_subcores=NS
    ),
    scratch_types=[
        pltpu.VMEM((NB, IDXPAD), jnp.int32),
        pltpu.VMEM((NB, CTX, EMBED), jnp.float32),
        pltpu.VMEM((CTX, EMBED), jnp.float32),
        pltpu.SemaphoreType.DMA((NB,)),
        pltpu.SemaphoreType.DMA((NB,)),
    ],
    compiler_params=pltpu.CompilerParams(use_tc_tiling_on_sc=False),
)
def _gather_add_pe(x_hbm, pe_hbm, table_hbm, out_hbm, idx_v, rows_v, pe_v, gsem, ssem):
    wid = lax.axis_index("s") * NC + lax.axis_index("c")
    pltpu.sync_copy(pe_hbm, pe_v)
    base = wid * ITEMS_PER_W

    def issue_gather(item, b):
        row0 = (base + item) * CTX
        pltpu.sync_copy(x_hbm.at[pl.ds(row0, CTX)], idx_v.at[b, pl.ds(0, CTX)])
        for c in range(IDXPAD // L):
            sl = pl.ds(c * L, L)
            t = idx_v[b, sl]
            j = (
                (t & jnp.int32(~(2 * DW - 1)))
                + ((t & jnp.int32(DW - 1)) << 1)
                + ((t >> LOG2DW) & jnp.int32(1))
            )
            idx_v[b, sl] = j
        pltpu.async_copy(
            table_hbm.at[idx_v.at[b, pl.ds(0, 128)]],
            rows_v.at[b, pl.ds(0, 128)],
            gsem.at[b],
        )
        pltpu.async_copy(
            table_hbm.at[idx_v.at[b, pl.ds(128, CTX - 128)]],
            rows_v.at[b, pl.ds(128, CTX - 128)],
            gsem.at[b],
        )

    def wait_gather(b):
        pltpu.make_async_copy(
            table_hbm.at[idx_v.at[b, pl.ds(0, CTX)]], rows_v.at[b], gsem.at[b]
        ).wait()

    def wait_store(b):
        pltpu.make_async_copy(
            rows_v.at[b], out_hbm.at[pl.ds(0, CTX), pl.ds(0, EMBED)], ssem.at[b]
        ).wait()

    def add_pe(b):
        def body(r, carry):
            for rr in range(2):
                for c in range(EMBED // L):
                    sl = pl.ds(c * L, L)
                    plsc.addupdate(rows_v.at[b, 2 * r + rr, sl], pe_v[2 * r + rr, sl])
            return carry

        lax.fori_loop(0, CTX // 2, body, 0)

    def issue_store(item, b):
        row0 = (base + item) * CTX
        pltpu.async_copy(
            rows_v.at[b], out_hbm.at[pl.ds(row0, CTX), pl.ds(0, EMBED)], ssem.at[b]
        )

    def process(item, b, do_wait_store, do_issue_gather):
        wait_gather(b)
        add_pe(b)
        issue_store(item, b)
        bn = (b + D) % NB
        if do_issue_gather:
            if do_wait_store:
                wait_store(bn)
            issue_gather(item + D, bn)

    for b in range(D):
        issue_gather(b, b)

    for b in range(NB):
        process(b, b, do_wait_store=(b >= NB - D), do_issue_gather=True)

    def group(j, carry):
        i0 = j * NB
        for b in range(NB):
            process(i0 + b, b, do_wait_store=True, do_issue_gather=True)
        return carry

    lax.fori_loop(1, GROUPS - 1, group, 0)

    i0 = (GROUPS - 1) * NB
    for b in range(NB):
        process(
            i0 + b, b, do_wait_store=True, do_issue_gather=(i0 + b + D < ITEMS_PER_W)
        )

    for b in range(NB):
        wait_store(b)


def kernel(x, table):
    pe = _pe_table()
    xf = x.reshape(-1).astype(jnp.int32)
    tt = table.T
    table_lin = _detile(tt, tt).reshape(VOCAB_PAD, EMBED)
    out = _gather_add_pe(xf, pe, table_lin)
    return out[:, :EMBED].reshape(BATCH, CTX, EMBED)


# speedup vs baseline: 2.1286x; 1.1260x over previous
import functools

import jax
import jax.numpy as jnp
from jax import lax
from jax.experimental import pallas as pl
from jax.experimental.pallas import tpu as pltpu
from jax.experimental.pallas import tpu_sc as plsc

VOCAB = 1000000
EMBED = 64
CTX = 200
BATCH = 4096

NC, NS = 2, 16
NW = NC * NS
ITEMS_PER_W = BATCH // NW
L = 16

NB = 8
D = 4
GROUPS = ITEMS_PER_W // NB
IDXPAD = 208

DW = 4096
LOG2DW = DW.bit_length() - 1
DGRID = (VOCAB + 2 * DW - 1) // (2 * DW)
VOCAB_PAD = DGRID * 2 * DW


def _pe_table():
    positions = jnp.arange(CTX, dtype=jnp.float32)
    dimensions = jnp.arange(EMBED, dtype=jnp.float32)
    exponent = (dimensions // 2) * 2.0 / EMBED
    divisor = jnp.power(10000.0, exponent)
    angle_rates = positions[:, None] / divisor
    pe = jnp.zeros_like(angle_rates)
    pe = pe.at[:, 0::2].set(jnp.sin(angle_rates[:, 0::2]))
    pe = pe.at[:, 1::2].set(jnp.cos(angle_rates[:, 1::2]))
    return pe


def _detile_body(a_ref, b_ref, o_ref):
    o_ref[...] = jnp.concatenate([a_ref[...], b_ref[...]], axis=0).T


_detile = pl.pallas_call(
    _detile_body,
    out_shape=jax.ShapeDtypeStruct((VOCAB_PAD // 2, 2 * EMBED), jnp.float32),
    grid=(DGRID,),
    in_specs=[
        pl.BlockSpec((EMBED, DW), lambda g: (0, 2 * g)),
        pl.BlockSpec(
            (EMBED, DW), lambda g: (0, jnp.minimum(2 * g + 1, VOCAB // DW - 1))
        ),
    ],
    out_specs=pl.BlockSpec((DW, 2 * EMBED), lambda g: (g, 0)),
)


@functools.partial(
    pl.kernel,
    out_type=jax.ShapeDtypeStruct((BATCH * CTX, 2 * EMBED), jnp.float32),
    mesh=plsc.VectorSubcoreMesh(
        core_axis_name="c", subcore_axis_name="s", num_cores=NC, num_subcores=NS
    ),
    scratch_types=[
        pltpu.VMEM((NB, IDXPAD), jnp.int32),
        pltpu.VMEM((NB, CTX, EMBED), jnp.float32),
        pltpu.VMEM((CTX, EMBED), jnp.float32),
        pltpu.SemaphoreType.DMA((NB,)),
        pltpu.SemaphoreType.DMA((NB,)),
    ],
    compiler_params=pltpu.CompilerParams(use_tc_tiling_on_sc=False),
)
def _gather_add_pe(x_hbm, pe_hbm, table_hbm, out_hbm, idx_v, rows_v, pe_v, gsem, ssem):
    wid = lax.axis_index("s") * NC + lax.axis_index("c")
    pltpu.sync_copy(pe_hbm, pe_v)
    base = wid * ITEMS_PER_W

    def issue_gather(item, b):
        row0 = (base + item) * CTX
        pltpu.sync_copy(x_hbm.at[pl.ds(row0, CTX)], idx_v.at[b, pl.ds(0, CTX)])
        for c in range(IDXPAD // L):
            sl = pl.ds(c * L, L)
            t = idx_v[b, sl]
            j = (
                (t & jnp.int32(~(2 * DW - 1)))
                + ((t & jnp.int32(DW - 1)) << 1)
                + ((t >> LOG2DW) & jnp.int32(1))
            )
            idx_v[b, sl] = j
        pltpu.async_copy(
            table_hbm.at[idx_v.at[b, pl.ds(0, 128)]],
            rows_v.at[b, pl.ds(0, 128)],
            gsem.at[b],
        )
        pltpu.async_copy(
            table_hbm.at[idx_v.at[b, pl.ds(128, CTX - 128)]],
            rows_v.at[b, pl.ds(128, CTX - 128)],
            gsem.at[b],
        )

    def wait_gather(b):
        pltpu.make_async_copy(
            table_hbm.at[idx_v.at[b, pl.ds(0, CTX)]], rows_v.at[b], gsem.at[b]
        ).wait()

    def wait_store(b):
        pltpu.make_async_copy(
            rows_v.at[b], out_hbm.at[pl.ds(0, CTX), pl.ds(0, EMBED)], ssem.at[b]
        ).wait()

    def add_pe(b):
        def body(r, carry):
            for rr in range(2):
                for c in range(EMBED // L):
                    sl = pl.ds(c * L, L)
                    plsc.addupdate(rows_v.at[b, 2 * r + rr, sl], pe_v[2 * r + rr, sl])
            return carry

        lax.fori_loop(0, CTX // 2, body, 0)

    def issue_store(item, b):
        row0 = (base + item) * CTX
        pltpu.async_copy(
            rows_v.at[b], out_hbm.at[pl.ds(row0, CTX), pl.ds(0, EMBED)], ssem.at[b]
        )

    def process(item, b, do_wait_store, do_issue_gather):
        wait_gather(b)
        add_pe(b)
        issue_store(item, b)
        bn = (b + D) % NB
        if do_issue_gather:
            if do_wait_store:
                wait_store(bn)
            issue_gather(item + D, bn)

    for b in range(D):
        issue_gather(b, b)

    for b in range(NB):
        process(b, b, do_wait_store=(b >= NB - D), do_issue_gather=True)

    def group(j, carry):
        i0 = j * NB
        for b in range(NB):
            process(i0 + b, b, do_wait_store=True, do_issue_gather=True)
        return carry

    lax.fori_loop(1, GROUPS - 1, group, 0)

    i0 = (GROUPS - 1) * NB
    for b in range(NB):
        process(
            i0 + b, b, do_wait_store=True, do_issue_gather=(i0 + b + D < ITEMS_PER_W)
        )

    for b in range(NB):
        wait_store(b)


def kernel(x, table):
    pe = _pe_table()
    xf = x.reshape(-1).astype(jnp.int32)
    tt = table.T
    table_lin = _detile(tt, tt).reshape(VOCAB_PAD, EMBED)
    out = _gather_add_pe(xf, pe, table_lin)
    return out[:, :EMBED].reshape(BATCH, CTX, EMBED)


# speedup vs baseline: 2.2309x; 1.0480x over previous
import functools

import jax
import jax.numpy as jnp
from jax import lax
from jax.experimental import pallas as pl
from jax.experimental.pallas import tpu as pltpu
from jax.experimental.pallas import tpu_sc as plsc

VOCAB = 1000000
EMBED = 64
CTX = 200
BATCH = 4096

NC, NS = 2, 16
NW = NC * NS
ITEMS_PER_W = BATCH // NW
L = 16

NB = 8
D = 4
GROUPS = ITEMS_PER_W // NB
IDXPAD = 208

DW = 8192
LOG2DW = DW.bit_length() - 1
DGRID = (VOCAB + 2 * DW - 1) // (2 * DW)
VOCAB_PAD = DGRID * 2 * DW


def _pe_table():
    positions = jnp.arange(CTX, dtype=jnp.float32)
    dimensions = jnp.arange(EMBED, dtype=jnp.float32)
    exponent = (dimensions // 2) * 2.0 / EMBED
    divisor = jnp.power(10000.0, exponent)
    angle_rates = positions[:, None] / divisor
    pe = jnp.zeros_like(angle_rates)
    pe = pe.at[:, 0::2].set(jnp.sin(angle_rates[:, 0::2]))
    pe = pe.at[:, 1::2].set(jnp.cos(angle_rates[:, 1::2]))
    return pe


def _detile_body(a_ref, b_ref, o_ref):
    o_ref[...] = jnp.concatenate([a_ref[...], b_ref[...]], axis=0).T


_detile = pl.pallas_call(
    _detile_body,
    out_shape=jax.ShapeDtypeStruct((VOCAB_PAD // 2, 2 * EMBED), jnp.float32),
    grid=(DGRID,),
    in_specs=[
        pl.BlockSpec((EMBED, DW), lambda g: (0, 2 * g)),
        pl.BlockSpec(
            (EMBED, DW), lambda g: (0, jnp.minimum(2 * g + 1, VOCAB // DW - 1))
        ),
    ],
    out_specs=pl.BlockSpec((DW, 2 * EMBED), lambda g: (g, 0)),
)


@functools.partial(
    pl.kernel,
    out_type=jax.ShapeDtypeStruct((BATCH * CTX, 2 * EMBED), jnp.float32),
    mesh=plsc.VectorSubcoreMesh(
        core_axis_name="c", subcore_axis_name="s", num_cores=NC, num_subcores=NS
    ),
    scratch_types=[
        pltpu.VMEM((NB, IDXPAD), jnp.int32),
        pltpu.VMEM((NB, CTX, EMBED), jnp.float32),
        pltpu.VMEM((CTX, EMBED), jnp.float32),
        pltpu.SemaphoreType.DMA((NB,)),
        pltpu.SemaphoreType.DMA((NB,)),
    ],
    compiler_params=pltpu.CompilerParams(use_tc_tiling_on_sc=False),
)
def _gather_add_pe(x_hbm, pe_hbm, table_hbm, out_hbm, idx_v, rows_v, pe_v, gsem, ssem):
    wid = lax.axis_index("s") * NC + lax.axis_index("c")
    pltpu.sync_copy(pe_hbm, pe_v)
    base = wid * ITEMS_PER_W

    def issue_gather(item, b):
        row0 = (base + item) * CTX
        pltpu.sync_copy(x_hbm.at[pl.ds(row0, CTX)], idx_v.at[b, pl.ds(0, CTX)])
        for c in range(IDXPAD // L):
            sl = pl.ds(c * L, L)
            t = idx_v[b, sl]
            j = (
                (t & jnp.int32(~(2 * DW - 1)))
                + ((t & jnp.int32(DW - 1)) << 1)
                + ((t >> LOG2DW) & jnp.int32(1))
            )
            idx_v[b, sl] = j
        pltpu.async_copy(
            table_hbm.at[idx_v.at[b, pl.ds(0, 128)]],
            rows_v.at[b, pl.ds(0, 128)],
            gsem.at[b],
        )
        pltpu.async_copy(
            table_hbm.at[idx_v.at[b, pl.ds(128, CTX - 128)]],
            rows_v.at[b, pl.ds(128, CTX - 128)],
            gsem.at[b],
        )

    def wait_gather(b):
        pltpu.make_async_copy(
            table_hbm.at[idx_v.at[b, pl.ds(0, CTX)]], rows_v.at[b], gsem.at[b]
        ).wait()

    def wait_store(b):
        pltpu.make_async_copy(
            rows_v.at[b], out_hbm.at[pl.ds(0, CTX), pl.ds(0, EMBED)], ssem.at[b]
        ).wait()

    def add_pe(b):
        def body(r, carry):
            for rr in range(2):
                for c in range(EMBED // L):
                    sl = pl.ds(c * L, L)
                    plsc.addupdate(rows_v.at[b, 2 * r + rr, sl], pe_v[2 * r + rr, sl])
            return carry

        lax.fori_loop(0, CTX // 2, body, 0)

    def issue_store(item, b):
        row0 = (base + item) * CTX
        pltpu.async_copy(
            rows_v.at[b], out_hbm.at[pl.ds(row0, CTX), pl.ds(0, EMBED)], ssem.at[b]
        )

    def process(item, b, do_wait_store, do_issue_gather):
        wait_gather(b)
        add_pe(b)
        issue_store(item, b)
        bn = (b + D) % NB
        if do_issue_gather:
            if do_wait_store:
                wait_store(bn)
            issue_gather(item + D, bn)

    for b in range(D):
        issue_gather(b, b)

    for b in range(NB):
        process(b, b, do_wait_store=(b >= NB - D), do_issue_gather=True)

    def group(j, carry):
        i0 = j * NB
        for b in range(NB):
            process(i0 + b, b, do_wait_store=True, do_issue_gather=True)
        return carry

    lax.fori_loop(1, GROUPS - 1, group, 0)

    i0 = (GROUPS - 1) * NB
    for b in range(NB):
        process(
            i0 + b, b, do_wait_store=True, do_issue_gather=(i0 + b + D < ITEMS_PER_W)
        )

    for b in range(NB):
        wait_store(b)


def kernel(x, table):
    pe = _pe_table()
    xf = x.reshape(-1).astype(jnp.int32)
    tt = table.T
    table_lin = _detile(tt, tt).reshape(VOCAB_PAD, EMBED)
    out = _gather_add_pe(xf, pe, table_lin)
    return out[:, :EMBED].reshape(BATCH, CTX, EMBED)


# speedup vs baseline: 2.2379x; 1.0032x over previous
import functools

import jax
import jax.numpy as jnp
from jax import lax
from jax.experimental import pallas as pl
from jax.experimental.pallas import tpu as pltpu
from jax.experimental.pallas import tpu_sc as plsc

VOCAB = 1000000
EMBED = 64
CTX = 200
BATCH = 4096

NC, NS = 2, 16
NW = NC * NS
ITEMS_PER_W = BATCH // NW
L = 16

NB = 8
D = 4
GROUPS = ITEMS_PER_W // NB
IDXPAD = 208

DW = 16384
LOG2DW = DW.bit_length() - 1
DGRID = (VOCAB + 2 * DW - 1) // (2 * DW)
VOCAB_PAD = DGRID * 2 * DW


def _pe_table():
    positions = jnp.arange(CTX, dtype=jnp.float32)
    dimensions = jnp.arange(EMBED, dtype=jnp.float32)
    exponent = (dimensions // 2) * 2.0 / EMBED
    divisor = jnp.power(10000.0, exponent)
    angle_rates = positions[:, None] / divisor
    pe = jnp.zeros_like(angle_rates)
    pe = pe.at[:, 0::2].set(jnp.sin(angle_rates[:, 0::2]))
    pe = pe.at[:, 1::2].set(jnp.cos(angle_rates[:, 1::2]))
    return pe


def _detile_body(a_ref, b_ref, o_ref):
    o_ref[...] = jnp.concatenate([a_ref[...], b_ref[...]], axis=0).T


_detile = pl.pallas_call(
    _detile_body,
    out_shape=jax.ShapeDtypeStruct((VOCAB_PAD // 2, 2 * EMBED), jnp.float32),
    grid=(DGRID,),
    in_specs=[
        pl.BlockSpec((EMBED, DW), lambda g: (0, 2 * g)),
        pl.BlockSpec(
            (EMBED, DW), lambda g: (0, jnp.minimum(2 * g + 1, VOCAB // DW - 1))
        ),
    ],
    out_specs=pl.BlockSpec((DW, 2 * EMBED), lambda g: (g, 0)),
)


@functools.partial(
    pl.kernel,
    out_type=jax.ShapeDtypeStruct((BATCH * CTX, 2 * EMBED), jnp.float32),
    mesh=plsc.VectorSubcoreMesh(
        core_axis_name="c", subcore_axis_name="s", num_cores=NC, num_subcores=NS
    ),
    scratch_types=[
        pltpu.VMEM((NB, IDXPAD), jnp.int32),
        pltpu.VMEM((NB, CTX, EMBED), jnp.float32),
        pltpu.VMEM((CTX, EMBED), jnp.float32),
        pltpu.SemaphoreType.DMA((NB,)),
        pltpu.SemaphoreType.DMA((NB,)),
    ],
    compiler_params=pltpu.CompilerParams(use_tc_tiling_on_sc=False),
)
def _gather_add_pe(x_hbm, pe_hbm, table_hbm, out_hbm, idx_v, rows_v, pe_v, gsem, ssem):
    wid = lax.axis_index("s") * NC + lax.axis_index("c")
    pltpu.sync_copy(pe_hbm, pe_v)
    base = wid * ITEMS_PER_W

    def issue_gather(item, b):
        row0 = (base + item) * CTX
        pltpu.sync_copy(x_hbm.at[pl.ds(row0, CTX)], idx_v.at[b, pl.ds(0, CTX)])
        for c in range(IDXPAD // L):
            sl = pl.ds(c * L, L)
            t = idx_v[b, sl]
            j = (
                (t & jnp.int32(~(2 * DW - 1)))
                + ((t & jnp.int32(DW - 1)) << 1)
                + ((t >> LOG2DW) & jnp.int32(1))
            )
            idx_v[b, sl] = j
        pltpu.async_copy(
            table_hbm.at[idx_v.at[b, pl.ds(0, 128)]],
            rows_v.at[b, pl.ds(0, 128)],
            gsem.at[b],
        )
        pltpu.async_copy(
            table_hbm.at[idx_v.at[b, pl.ds(128, CTX - 128)]],
            rows_v.at[b, pl.ds(128, CTX - 128)],
            gsem.at[b],
        )

    def wait_gather(b):
        pltpu.make_async_copy(
            table_hbm.at[idx_v.at[b, pl.ds(0, CTX)]], rows_v.at[b], gsem.at[b]
        ).wait()

    def wait_store(b):
        pltpu.make_async_copy(
            rows_v.at[b], out_hbm.at[pl.ds(0, CTX), pl.ds(0, EMBED)], ssem.at[b]
        ).wait()

    def add_pe(b):
        def body(r, carry):
            for rr in range(2):
                for c in range(EMBED // L):
                    sl = pl.ds(c * L, L)
                    plsc.addupdate(rows_v.at[b, 2 * r + rr, sl], pe_v[2 * r + rr, sl])
            return carry

        lax.fori_loop(0, CTX // 2, body, 0)

    def issue_store(item, b):
        row0 = (base + item) * CTX
        pltpu.async_copy(
            rows_v.at[b], out_hbm.at[pl.ds(row0, CTX), pl.ds(0, EMBED)], ssem.at[b]
        )

    def process(item, b, do_wait_store, do_issue_gather):
        wait_gather(b)
        add_pe(b)
        issue_store(item, b)
        bn = (b + D) % NB
        if do_issue_gather:
            if do_wait_store:
                wait_store(bn)
            issue_gather(item + D, bn)

    for b in range(D):
        issue_gather(b, b)

    for b in range(NB):
        process(b, b, do_wait_store=(b >= NB - D), do_issue_gather=True)

    def group(j, carry):
        i0 = j * NB
        for b in range(NB):
            process(i0 + b, b, do_wait_store=True, do_issue_gather=True)
        return carry

    lax.fori_loop(1, GROUPS - 1, group, 0)

    i0 = (GROUPS - 1) * NB
    for b in range(NB):
        process(
            i0 + b, b, do_wait_store=True, do_issue_gather=(i0 + b + D < ITEMS_PER_W)
        )

    for b in range(NB):
        wait_store(b)


def kernel(x, table):
    pe = _pe_table()
    xf = x.reshape(-1).astype(jnp.int32)
    tt = table.T
    table_lin = _detile(tt, tt).reshape(VOCAB_PAD, EMBED)
    out = _gather_add_pe(xf, pe, table_lin)
    return out[:, :EMBED].reshape(BATCH, CTX, EMBED)
